# initial kernel scaffold (unmeasured)
import jax
import jax.numpy as jnp
from jax import lax
from jax.experimental import pallas as pl
from jax.experimental.pallas import tpu as pltpu


def kernel(
    x,
):
    def body(*refs):
        pass

    out_shape = jax.ShapeDtypeStruct(..., jnp.float32)
    return pl.pallas_call(body, out_shape=out_shape)(...)


# baseline (device time: 193240 ns/iter reference)
import jax
import jax.numpy as jnp
from jax import lax
from jax.experimental import pallas as pl
from jax.experimental.pallas import tpu as pltpu

N_DEV = 8


def kernel(x):
    _, m, n = x.shape
    ch = m // N_DEV

    def body(x_ref, out_ref, send_buf, recv_bufs, send_sems, recv_sems):
        my = lax.axis_index("i")
        left = (my - 1) % N_DEV
        right = (my + 1) % N_DEV

        barrier_sem = pltpu.get_barrier_semaphore()
        for nbr in (left, right):
            pl.semaphore_signal(
                barrier_sem,
                inc=1,
                device_id=(nbr,),
                device_id_type=pl.DeviceIdType.MESH,
            )
        pl.semaphore_wait(barrier_sem, 2)

        for s in range(N_DEV - 1):
            c = (my - s) % N_DEV
            row = c * ch
            if s == 0:
                send_buf[...] = x_ref[0, pl.ds(row, ch), :]
            else:
                send_buf[...] = recv_bufs[s - 1] + x_ref[0, pl.ds(row, ch), :]
            rdma = pltpu.make_async_remote_copy(
                src_ref=send_buf,
                dst_ref=recv_bufs.at[s],
                send_sem=send_sems.at[s],
                recv_sem=recv_sems.at[s],
                device_id=(right,),
                device_id_type=pl.DeviceIdType.MESH,
            )
            rdma.start()
            rdma.wait()

        r = (my + 1) % N_DEV
        rrow = r * ch
        out_ref[pl.ds(rrow, ch), :] = (
            recv_bufs[N_DEV - 2] + x_ref[0, pl.ds(rrow, ch), :]
        )

        for s in range(N_DEV - 1):
            c = (my + 1 - s) % N_DEV
            row = c * ch
            rdma = pltpu.make_async_remote_copy(
                src_ref=out_ref.at[pl.ds(row, ch), :],
                dst_ref=out_ref.at[pl.ds(row, ch), :],
                send_sem=send_sems.at[N_DEV - 1 + s],
                recv_sem=recv_sems.at[N_DEV - 1 + s],
                device_id=(right,),
                device_id_type=pl.DeviceIdType.MESH,
            )
            rdma.start()
            rdma.wait()

    n_steps = 2 * (N_DEV - 1)
    return pl.pallas_call(
        body,
        out_shape=jax.ShapeDtypeStruct((m, n), x.dtype),
        in_specs=[pl.BlockSpec(memory_space=pltpu.VMEM)],
        out_specs=pl.BlockSpec(memory_space=pltpu.VMEM),
        scratch_shapes=[
            pltpu.VMEM((ch, n), x.dtype),
            pltpu.VMEM((N_DEV - 1, ch, n), x.dtype),
            pltpu.SemaphoreType.DMA((n_steps,)),
            pltpu.SemaphoreType.DMA((n_steps,)),
        ],
        compiler_params=pltpu.CompilerParams(collective_id=0),
    )(x)
